# Initial kernel scaffold; baseline (speedup 1.0000x reference)
import jax
import jax.numpy as jnp
from jax.experimental import pallas as pl


def kernel(x, anchor_ids, anchor_alignment, table, W, gate):
    raise NotImplementedError("write your pallas kernel here")


# speedup vs baseline: 8.2810x; 8.2810x over previous
import functools

import jax
import jax.numpy as jnp
from jax import lax
from jax.experimental import pallas as pl
from jax.experimental.pallas import tpu as pltpu
from jax.experimental.pallas import tpu_sc as plsc

NC = 2
NS = 16
NW = NC * NS
IDX_W = 128
FIRE = 10


def _sc_gather(ids3d, table, n_rows, d):
    rows_per_w = n_rows // NW
    idx_rows = rows_per_w // IDX_W
    n_batches = idx_rows // FIRE
    buf_rows = FIRE * IDX_W

    mesh = plsc.VectorSubcoreMesh(core_axis_name="c", subcore_axis_name="s")

    @functools.partial(
        pl.kernel,
        mesh=mesh,
        compiler_params=pltpu.CompilerParams(use_tc_tiling_on_sc=False),
        out_type=jax.ShapeDtypeStruct((n_rows, d), jnp.float32),
        scratch_types=[
            pltpu.VMEM((idx_rows, IDX_W), jnp.int32),
            pltpu.VMEM((buf_rows, d), jnp.float32),
            pltpu.SemaphoreType.DMA,
        ],
    )
    def gather_kernel(ids_hbm, table_hbm, out_hbm, idx_v, rows_v, sem):
        wid = lax.axis_index("s") * NC + lax.axis_index("c")
        pltpu.sync_copy(ids_hbm.at[wid], idx_v)
        for c in range(n_batches):
            handles = []
            for j in range(FIRE):
                handles.append(pltpu.async_copy(
                    table_hbm.at[idx_v.at[c * FIRE + j]],
                    rows_v.at[pl.ds(j * IDX_W, IDX_W)],
                    sem,
                ))
            for h in handles:
                h.wait()
            base = wid * rows_per_w + c * buf_rows
            pltpu.sync_copy(rows_v, out_hbm.at[pl.ds(base, buf_rows)])

    return gather_kernel(ids3d, table)


def _tc_body(na, bb, ll, xf_ref, align_ref, embs_ref, wt_ref, gate_ref, out_ref):
    proj = jnp.dot(embs_ref[...], wt_ref[...], preferred_element_type=jnp.float32)
    align = align_ref[...].reshape(1, bb * ll)
    r = lax.broadcasted_iota(jnp.int32, (1, bb * ll), 1)
    gidx = align + (r // ll) * na
    k = lax.broadcasted_iota(jnp.int32, (bb * na, bb * ll), 0)
    onehot_t = (k == gidx).astype(jnp.float32)
    sel = lax.dot_general(onehot_t, proj, (((0,), (0,)), ((), ())),
                          preferred_element_type=jnp.float32)
    t = jnp.tanh(gate_ref[0, 0])
    out_ref[...] = xf_ref[...] + t * sel


def kernel(x, anchor_ids, anchor_alignment, table, W, gate):
    b, ll, d = x.shape
    na = anchor_ids.shape[1]
    bb = 8

    ids3d = anchor_ids.astype(jnp.int32).reshape(NW, b * na // (NW * IDX_W), IDX_W)
    anchor_embs = _sc_gather(ids3d, table, b * na, d)

    xf = x.reshape(b * ll, d)
    align2 = anchor_alignment.astype(jnp.int32).reshape(b // bb, 1, bb * ll)
    wt = W.T
    gate2 = gate.reshape(1, 1)
    nblk = b // bb

    out = pl.pallas_call(
        functools.partial(_tc_body, na, bb, ll),
        grid=(nblk,),
        in_specs=[
            pl.BlockSpec((bb * ll, d), lambda i: (i, 0)),
            pl.BlockSpec((1, 1, bb * ll), lambda i: (i, 0, 0)),
            pl.BlockSpec((bb * na, d), lambda i: (i, 0)),
            pl.BlockSpec((d, d), lambda i: (0, 0)),
            pl.BlockSpec((1, 1), lambda i: (0, 0)),
        ],
        out_specs=pl.BlockSpec((bb * ll, d), lambda i: (i, 0)),
        out_shape=jax.ShapeDtypeStruct((b * ll, d), jnp.float32),
    )(xf, align2, anchor_embs, wt, gate2)

    return out.reshape(b, ll, d)


# speedup vs baseline: 11.3146x; 1.3663x over previous
import functools

import jax
import jax.numpy as jnp
from jax import lax
from jax.experimental import pallas as pl
from jax.experimental.pallas import tpu as pltpu
from jax.experimental.pallas import tpu_sc as plsc

NC = 2
NS = 16
NW = NC * NS
IDX_W = 128
FIRE = 10


def _sc_gather(ids3d, table, n_rows, d):
    rows_per_w = n_rows // NW
    idx_rows = rows_per_w // IDX_W
    n_batches = idx_rows // FIRE
    buf_rows = FIRE * IDX_W

    mesh = plsc.VectorSubcoreMesh(core_axis_name="c", subcore_axis_name="s")

    @functools.partial(
        pl.kernel,
        mesh=mesh,
        compiler_params=pltpu.CompilerParams(use_tc_tiling_on_sc=False),
        out_type=jax.ShapeDtypeStruct((n_rows, d), jnp.float32),
        scratch_types=[
            pltpu.VMEM((idx_rows, IDX_W), jnp.int32),
            pltpu.VMEM((buf_rows, d), jnp.float32),
            pltpu.SemaphoreType.DMA,
        ],
    )
    def gather_kernel(ids_hbm, table_hbm, out_hbm, idx_v, rows_v, sem):
        wid = lax.axis_index("s") * NC + lax.axis_index("c")
        pltpu.sync_copy(ids_hbm.at[wid], idx_v)
        for c in range(n_batches):
            handles = []
            for j in range(FIRE):
                handles.append(pltpu.async_copy(
                    table_hbm.at[idx_v.at[c * FIRE + j]],
                    rows_v.at[pl.ds(j * IDX_W, IDX_W)],
                    sem,
                ))
            for h in handles:
                h.wait()
            base = wid * rows_per_w + c * buf_rows
            pltpu.sync_copy(rows_v, out_hbm.at[pl.ds(base, buf_rows)])

    return gather_kernel(ids3d, table)


def _tc_body(na, bb, lb, d, xt_ref, alignt_ref, embs_ref, w_ref, gate_ref, out_ref):
    embs2 = embs_ref[...].reshape(na * bb, d)
    proj_all = lax.dot_general(w_ref[...], embs2, (((1,), (1,)), ((), ())),
                               preferred_element_type=jnp.float32)
    align = alignt_ref[...]
    acc = jnp.broadcast_to(proj_all[:, 0:bb][None, :, :], (lb, d, bb))
    for a in range(1, na):
        pa = proj_all[:, a * bb:(a + 1) * bb]
        acc = jnp.where((align == a)[:, None, :], pa[None, :, :], acc)
    t = jnp.tanh(gate_ref[0, 0])
    out_ref[...] = xt_ref[...] + t * acc


def kernel(x, anchor_ids, anchor_alignment, table, W, gate):
    b, ll, d = x.shape
    na = anchor_ids.shape[1]
    bb = 128
    lb = 40

    ids3d = anchor_ids.astype(jnp.int32).T.reshape(NW, b * na // (NW * IDX_W), IDX_W)
    anchor_embs = _sc_gather(ids3d, table, b * na, d)
    embs3 = anchor_embs.reshape(na, b, d)

    xt = jnp.transpose(x, (1, 2, 0))
    alignt = anchor_alignment.astype(jnp.int32).T
    gate2 = gate.reshape(1, 1)

    outt = pl.pallas_call(
        functools.partial(_tc_body, na, bb, lb, d),
        grid=(b // bb, ll // lb),
        in_specs=[
            pl.BlockSpec((lb, d, bb), lambda i, j: (j, 0, i)),
            pl.BlockSpec((lb, bb), lambda i, j: (j, i)),
            pl.BlockSpec((na, bb, d), lambda i, j: (0, i, 0)),
            pl.BlockSpec((d, d), lambda i, j: (0, 0)),
            pl.BlockSpec((1, 1), lambda i, j: (0, 0)),
        ],
        out_specs=pl.BlockSpec((lb, d, bb), lambda i, j: (j, 0, i)),
        out_shape=jax.ShapeDtypeStruct((ll, d, b), jnp.float32),
    )(xt, alignt, embs3, W, gate2)

    return jnp.transpose(outt, (2, 0, 1))
